# baseline (device time: 47078 ns/iter reference)
import jax
import jax.numpy as jnp
from jax import lax
from jax.experimental import pallas as pl
from jax.experimental.pallas import tpu as pltpu

N_DEV = 4
N_TOK = 512
D_IN = 256
D_OUT = 512
E_PER = 2


def kernel(x, router_W, route_idx, expert_W):
    def body(x_ref, rw_ref, idx_ref, ew_ref, out_ref, comm_ref, send_sems, recv_sems):
        my_pos = lax.axis_index("i")
        left = lax.rem(my_pos + N_DEV - 1, N_DEV)
        right = lax.rem(my_pos + 1, N_DEV)

        barrier_sem = pltpu.get_barrier_semaphore()
        for nbr in (left, right):
            pl.semaphore_signal(
                barrier_sem, inc=1,
                device_id=(nbr,), device_id_type=pl.DeviceIdType.MESH,
            )
        pl.semaphore_wait(barrier_sem, 2)

        idx = idx_ref[:, :]
        e0 = my_pos * E_PER
        m0 = (idx == e0).astype(jnp.float32)
        m1 = (idx == e0 + 1).astype(jnp.float32)
        xm = jnp.concatenate([x_ref[:, :] * m0, x_ref[:, :] * m1], axis=1)
        w = ew_ref[:, :, :].reshape(E_PER * D_IN, D_OUT)
        part = jnp.dot(xm, w, preferred_element_type=jnp.float32)

        out_ref[:, :] = part
        comm_ref[0, :, :] = part

        for h in range(N_DEV - 1):
            rdma = pltpu.make_async_remote_copy(
                src_ref=comm_ref.at[h],
                dst_ref=comm_ref.at[h + 1],
                send_sem=send_sems.at[h],
                recv_sem=recv_sems.at[h],
                device_id=(right,),
                device_id_type=pl.DeviceIdType.MESH,
            )
            rdma.start()
            rdma.wait()
            out_ref[:, :] += comm_ref[h + 1, :, :]

    return pl.pallas_call(
        body,
        out_shape=jax.ShapeDtypeStruct((N_TOK, D_OUT), jnp.float32),
        in_specs=[
            pl.BlockSpec(memory_space=pltpu.VMEM),
            pl.BlockSpec(memory_space=pltpu.VMEM),
            pl.BlockSpec(memory_space=pltpu.VMEM),
            pl.BlockSpec(memory_space=pltpu.VMEM),
        ],
        out_specs=pl.BlockSpec(memory_space=pltpu.VMEM),
        scratch_shapes=[
            pltpu.VMEM((N_DEV, N_TOK, D_OUT), jnp.float32),
            pltpu.SemaphoreType.DMA((N_DEV - 1,)),
            pltpu.SemaphoreType.DMA((N_DEV - 1,)),
        ],
        compiler_params=pltpu.CompilerParams(collective_id=0),
    )(x, router_W, route_idx, expert_W)


# device time: 31795 ns/iter; 1.4807x vs baseline; 1.4807x over previous
import jax
import jax.numpy as jnp
from jax import lax
from jax.experimental import pallas as pl
from jax.experimental.pallas import tpu as pltpu

N_DEV = 4
N_TOK = 512
D_IN = 256
D_OUT = 512
E_PER = 2
HALF = D_OUT // 2
QUART = D_OUT // 4


def kernel(x, router_W, route_idx, expert_W):
    def body(x_ref, rw_ref, idx_ref, ew_ref, out_ref,
             s1, r1, s2, r2, s3, r3, s4, r4, acc1, send_sems, recv_sems):
        p = lax.axis_index("i")
        partner1 = p ^ 1
        partner2 = p ^ 3

        in_right = jnp.logical_or(p == 1, p == 2)
        keep_off1 = jnp.where(in_right, HALF, 0)
        send_off1 = HALF - keep_off1
        rel_keep2 = jnp.where(p >= 2, QUART, 0)
        rel_send2 = QUART - rel_keep2
        q_abs = keep_off1 + rel_keep2

        barrier_sem = pltpu.get_barrier_semaphore()
        for nbr in (partner1, partner2):
            pl.semaphore_signal(
                barrier_sem, inc=1,
                device_id=(nbr,), device_id_type=pl.DeviceIdType.MESH,
            )
        pl.semaphore_wait(barrier_sem, 2)

        def xchg(step, src, dst, partner):
            return pltpu.make_async_remote_copy(
                src_ref=src, dst_ref=dst,
                send_sem=send_sems.at[step], recv_sem=recv_sems.at[step],
                device_id=(partner,), device_id_type=pl.DeviceIdType.MESH,
            )

        idx = idx_ref[:, :]
        e0 = p * E_PER
        m0 = (idx == e0).astype(jnp.float32)
        m1 = (idx == e0 + 1).astype(jnp.float32)
        xm = jnp.concatenate([x_ref[:, :] * m0, x_ref[:, :] * m1], axis=1)

        w_send = ew_ref[:, :, pl.ds(send_off1, HALF)].reshape(E_PER * D_IN, HALF)
        s1[:, :] = jnp.dot(xm, w_send, preferred_element_type=jnp.float32)
        rdma1 = xchg(0, s1, r1, partner1)
        rdma1.start()

        w_keep = ew_ref[:, :, pl.ds(keep_off1, HALF)].reshape(E_PER * D_IN, HALF)
        part_keep = jnp.dot(xm, w_keep, preferred_element_type=jnp.float32)
        rdma1.wait()
        acc1[:, :] = part_keep + r1[:, :]

        s2[:, :] = acc1[:, pl.ds(rel_send2, QUART)]
        rdma2 = xchg(1, s2, r2, partner2)
        rdma2.start()
        rdma2.wait()
        acc2 = acc1[:, pl.ds(rel_keep2, QUART)] + r2[:, :]

        s3[:, :] = acc2
        rdma3 = xchg(2, s3, r3, partner2)
        rdma3.start()
        rdma3.wait()
        s4[:, pl.ds(rel_keep2, QUART)] = acc2
        s4[:, pl.ds(rel_send2, QUART)] = r3[:, :]
        out_ref[:, pl.ds(keep_off1, HALF)] = s4[:, :]

        rdma4 = xchg(3, s4, r4, partner1)
        rdma4.start()
        rdma4.wait()
        out_ref[:, pl.ds(send_off1, HALF)] = r4[:, :]

    return pl.pallas_call(
        body,
        out_shape=jax.ShapeDtypeStruct((N_TOK, D_OUT), jnp.float32),
        in_specs=[
            pl.BlockSpec(memory_space=pltpu.VMEM),
            pl.BlockSpec(memory_space=pltpu.VMEM),
            pl.BlockSpec(memory_space=pltpu.VMEM),
            pl.BlockSpec(memory_space=pltpu.VMEM),
        ],
        out_specs=pl.BlockSpec(memory_space=pltpu.VMEM),
        scratch_shapes=[
            pltpu.VMEM((N_TOK, HALF), jnp.float32),
            pltpu.VMEM((N_TOK, HALF), jnp.float32),
            pltpu.VMEM((N_TOK, QUART), jnp.float32),
            pltpu.VMEM((N_TOK, QUART), jnp.float32),
            pltpu.VMEM((N_TOK, QUART), jnp.float32),
            pltpu.VMEM((N_TOK, QUART), jnp.float32),
            pltpu.VMEM((N_TOK, HALF), jnp.float32),
            pltpu.VMEM((N_TOK, HALF), jnp.float32),
            pltpu.VMEM((N_TOK, HALF), jnp.float32),
            pltpu.SemaphoreType.DMA((4,)),
            pltpu.SemaphoreType.DMA((4,)),
        ],
        compiler_params=pltpu.CompilerParams(collective_id=0),
    )(x, router_W, route_idx, expert_W)


# device time: 15100 ns/iter; 3.1177x vs baseline; 2.1056x over previous
import jax
import jax.numpy as jnp
from jax import lax
from jax.experimental import pallas as pl
from jax.experimental.pallas import tpu as pltpu

N_DEV = 4
N_TOK = 512
D_IN = 256
D_OUT = 512
E_PER = 2
QUART = D_OUT // 4
NC = 4
R = N_TOK // NC

BF16 = jnp.bfloat16
F32 = jnp.float32


def kernel(x, router_W, route_idx, expert_W):
    def body(x_ref, rw_ref, idx_ref, ew_ref, out_ref,
             sQ, rQ, sB, rB, send1, recv1, send2, recv2):
        p = lax.axis_index("i")
        peers = [p ^ 2, p ^ 1, p ^ 3]

        def q_off(t):
            right = jnp.logical_or(t == 1, t == 2)
            return jnp.where(right, 2 * QUART, 0) + jnp.where(t >= 2, QUART, 0)

        barrier_sem = pltpu.get_barrier_semaphore()
        for t in peers:
            pl.semaphore_signal(
                barrier_sem, inc=1,
                device_id=(t,), device_id_type=pl.DeviceIdType.MESH,
            )
        pl.semaphore_wait(barrier_sem, 3)

        def masked(c):
            xc = x_ref[pl.ds(c * R, R), :]
            idc = idx_ref[pl.ds(c * R, R), :]
            m0 = (idc == p * E_PER).astype(F32)
            m1 = (idc == p * E_PER + 1).astype(F32)
            return xc * m0, xc * m1

        xms = [masked(c) for c in range(NC)]

        def part(c, off):
            w0 = ew_ref[0, :, pl.ds(off, QUART)]
            w1 = ew_ref[1, :, pl.ds(off, QUART)]
            return (jnp.dot(xms[c][0], w0, preferred_element_type=F32)
                    + jnp.dot(xms[c][1], w1, preferred_element_type=F32))

        X1 = []
        for c in range(NC):
            for t in peers:
                sQ[t, c, :, :] = part(c, q_off(t)).astype(BF16)
                rd = pltpu.make_async_remote_copy(
                    src_ref=sQ.at[t, c], dst_ref=rQ.at[p, c],
                    send_sem=send1.at[t, c], recv_sem=recv1.at[p, c],
                    device_id=(t,), device_id_type=pl.DeviceIdType.MESH,
                )
                rd.start()
                X1.append(rd)

        mine = [part(c, q_off(p)) for c in range(NC)]

        X2 = []
        for c in range(NC):
            for s in peers:
                pltpu.make_async_remote_copy(
                    src_ref=sQ.at[s, c], dst_ref=rQ.at[s, c],
                    send_sem=send1.at[s, c], recv_sem=recv1.at[s, c],
                    device_id=(p,), device_id_type=pl.DeviceIdType.MESH,
                ).wait_recv()
            qc = (mine[c] + rQ[peers[0], c, :, :].astype(F32)
                  + rQ[peers[1], c, :, :].astype(F32)
                  + rQ[peers[2], c, :, :].astype(F32))
            sB[c, :, :] = qc.astype(BF16)
            for t in peers:
                rd = pltpu.make_async_remote_copy(
                    src_ref=sB.at[c], dst_ref=rB.at[p, c],
                    send_sem=send2.at[t, c], recv_sem=recv2.at[p, c],
                    device_id=(t,), device_id_type=pl.DeviceIdType.MESH,
                )
                rd.start()
                X2.append(rd)
            out_ref[pl.ds(c * R, R), pl.ds(q_off(p), QUART)] = qc

        for c in range(NC):
            for s in peers:
                pltpu.make_async_remote_copy(
                    src_ref=sB.at[c], dst_ref=rB.at[s, c],
                    send_sem=send2.at[s, c], recv_sem=recv2.at[s, c],
                    device_id=(p,), device_id_type=pl.DeviceIdType.MESH,
                ).wait_recv()
                out_ref[pl.ds(c * R, R), pl.ds(q_off(s), QUART)] = (
                    rB[s, c, :, :].astype(F32)
                )

        for rd in X1 + X2:
            rd.wait_send()

    return pl.pallas_call(
        body,
        out_shape=jax.ShapeDtypeStruct((N_TOK, D_OUT), jnp.float32),
        in_specs=[
            pl.BlockSpec(memory_space=pltpu.VMEM),
            pl.BlockSpec(memory_space=pltpu.VMEM),
            pl.BlockSpec(memory_space=pltpu.VMEM),
            pl.BlockSpec(memory_space=pltpu.VMEM),
        ],
        out_specs=pl.BlockSpec(memory_space=pltpu.VMEM),
        scratch_shapes=[
            pltpu.VMEM((N_DEV, NC, R, QUART), BF16),
            pltpu.VMEM((N_DEV, NC, R, QUART), BF16),
            pltpu.VMEM((NC, R, QUART), BF16),
            pltpu.VMEM((N_DEV, NC, R, QUART), BF16),
            pltpu.SemaphoreType.DMA((N_DEV, NC)),
            pltpu.SemaphoreType.DMA((N_DEV, NC)),
            pltpu.SemaphoreType.DMA((N_DEV, NC)),
            pltpu.SemaphoreType.DMA((N_DEV, NC)),
        ],
        compiler_params=pltpu.CompilerParams(collective_id=0),
    )(x, router_W, route_idx, expert_W)
